# baseline (speedup 1.0000x reference)
import functools

import jax
import jax.numpy as jnp
from jax import lax
from jax.experimental import pallas as pl
from jax.experimental.pallas import tpu as pltpu
from jax.experimental.pallas import tpu_sc as plsc

N = 10000
E = 320000
D = 128

NC = 2
NS = 16
NW = NC * NS
NPAD = 10240
K = 128
CH_PAIR = 160
EPAD = NS * CH_PAIR * K
N0 = 160
N1 = CH_PAIR - N0
PB = 2
PG = 1
PS = PB - PG
RPT = NPAD // NS


def _sc_segment_body(with_cnt, y_hbm, src_hbm, dst_hbm, *refs):
    if with_cnt:
        (s_hbm, cnt_hbm,
         acc, cntacc, sidx, didx, rows, ones, zcnt, gsem, ssem) = refs
    else:
        (s_hbm,
         acc, sidx, didx, rows, gsem, ssem) = refs
    cid = lax.axis_index("c")
    sid = lax.axis_index("s")
    myrows = pl.ds(sid * RPT, RPT)

    z16 = jnp.zeros((16,), jnp.float32)

    def _zrow(i, _):
        rows[0, i // (D // 16), pl.ds(lax.rem(i, D // 16) * 16, 16)] = z16
        return 0

    lax.fori_loop(0, K * D // 16, _zrow, 0)
    for j in range(RPT // K):
        pltpu.sync_copy(rows.at[0], acc.at[pl.ds(sid * RPT + j * K, K)])
    if with_cnt:
        def _zone(i, _):
            ones[pl.ds(i * 16, 16)] = jnp.ones((16,), jnp.float32)
            return 0

        lax.fori_loop(0, K // 16, _zone, 0)

        def _zc(i, _):
            zcnt[pl.ds(i * 16, 16)] = z16
            return 0

        lax.fori_loop(0, RPT // 16, _zc, 0)
        pltpu.sync_copy(zcnt, cntacc.at[myrows])
    plsc.subcore_barrier()

    nch = jnp.where(cid == 0, N0, N1)
    ebase = (sid * CH_PAIR + jnp.where(cid == 0, 0, N0)) * K

    def _gather(c, b):
        off = ebase + c * K
        pltpu.sync_copy(src_hbm.at[pl.ds(off, K)], sidx.at[b])
        pltpu.sync_copy(dst_hbm.at[pl.ds(off, K)], didx.at[b])
        pltpu.async_copy(y_hbm.at[sidx.at[b]], rows.at[b], gsem.at[b])

    def _wait_scatter(b):
        pltpu.make_async_copy(rows.at[b], acc.at[didx.at[b]],
                              ssem.at[b]).wait()
        if with_cnt:
            pltpu.make_async_copy(ones, cntacc.at[didx.at[b]],
                                  ssem.at[b]).wait()

    for i in range(PG):
        @pl.when(i < nch)
        def _():
            _gather(i, i)

    def _chunk(c, _):
        b = lax.rem(c, PB)

        @pl.when(c >= PS)
        def _():
            _wait_scatter(lax.rem(c - PS, PB))

        @pl.when(c + PG < nch)
        def _():
            _gather(c + PG, lax.rem(c + PG, PB))

        pltpu.make_async_copy(y_hbm.at[sidx.at[b]], rows.at[b],
                              gsem.at[b]).wait()
        pltpu.async_copy(rows.at[b], acc.at[didx.at[b]], ssem.at[b],
                         add=True)
        if with_cnt:
            pltpu.async_copy(ones, cntacc.at[didx.at[b]], ssem.at[b],
                             add=True)
        return 0

    lax.fori_loop(0, nch, _chunk, 0)
    for k in range(1, PS + 1):
        @pl.when(nch >= k)
        def _():
            _wait_scatter(lax.rem(nch - k + PB, PB))
    plsc.subcore_barrier()

    pltpu.sync_copy(acc.at[myrows], s_hbm.at[cid, myrows])
    if with_cnt:
        pltpu.sync_copy(cntacc.at[myrows], cnt_hbm.at[cid, myrows])


def _sc_segment_sum(y, src, dst, with_cnt):
    mesh = plsc.VectorSubcoreMesh(core_axis_name="c", subcore_axis_name="s")
    if with_cnt:
        out_type = (
            pltpu.HBM((NC, NPAD, D), jnp.float32),
            pltpu.HBM((NC, NPAD), jnp.float32),
        )
        scratch = [
            pltpu.VMEM_SHARED((NPAD, D), jnp.float32),
            pltpu.VMEM_SHARED((NPAD,), jnp.float32),
            pltpu.VMEM((PB, K), jnp.int32),
            pltpu.VMEM((PB, K), jnp.int32),
            pltpu.VMEM((PB, K, D), jnp.float32),
            pltpu.VMEM((K,), jnp.float32),
            pltpu.VMEM((RPT,), jnp.float32),
            pltpu.SemaphoreType.DMA((PB,)),
            pltpu.SemaphoreType.DMA((PB,)),
        ]
        args = (y, src, dst)
    else:
        out_type = pltpu.HBM((NC, NPAD, D), jnp.float32)
        scratch = [
            pltpu.VMEM_SHARED((NPAD, D), jnp.float32),
            pltpu.VMEM((PB, K), jnp.int32),
            pltpu.VMEM((PB, K), jnp.int32),
            pltpu.VMEM((PB, K, D), jnp.float32),
            pltpu.SemaphoreType.DMA((PB,)),
            pltpu.SemaphoreType.DMA((PB,)),
        ]
        args = (y, src, dst)
    return pl.kernel(
        functools.partial(_sc_segment_body, with_cnt),
        out_type=out_type,
        mesh=mesh,
        scratch_types=scratch,
    )(*args)


def _mm_body(x_ref, w_ref, o_ref):
    o_ref[...] = jnp.dot(x_ref[...], w_ref[...],
                         preferred_element_type=jnp.float32)


def _tc_matmul(x, wt):
    return pl.pallas_call(
        _mm_body,
        out_shape=jax.ShapeDtypeStruct((NPAD, D), jnp.float32),
    )(x, wt)


def _mid_body(s_ref, c_ref, x_ref, wr_ref, b_ref, wl2_ref, h_ref, y2_ref):
    s = s_ref[0] + s_ref[1]
    c = c_ref[0] + c_ref[1]
    agg = s / jnp.maximum(c, 1.0)
    h = jnp.maximum(
        agg + b_ref[...]
        + jnp.dot(x_ref[...], wr_ref[...], preferred_element_type=jnp.float32),
        0.0)
    h_ref[...] = h
    y2_ref[...] = jnp.dot(h, wl2_ref[...], preferred_element_type=jnp.float32)


def _tc_mid(s1, cnt, x, w1r_t, b1, w2l_t):
    return pl.pallas_call(
        _mid_body,
        out_shape=(
            jax.ShapeDtypeStruct((NPAD, D), jnp.float32),
            jax.ShapeDtypeStruct((NPAD, D), jnp.float32),
        ),
    )(s1, cnt, x, w1r_t, b1, w2l_t)


def _fin_body(s_ref, c_ref, h_ref, wr_ref, b_ref, o_ref):
    s = s_ref[0] + s_ref[1]
    c = c_ref[0] + c_ref[1]
    agg = s / jnp.maximum(c, 1.0)
    o_ref[...] = jnp.maximum(
        agg + b_ref[...]
        + jnp.dot(h_ref[...], wr_ref[...], preferred_element_type=jnp.float32),
        0.0)


def _tc_fin(s2, cnt, h, w2r_t, b2):
    return pl.pallas_call(
        _fin_body,
        out_shape=jax.ShapeDtypeStruct((NPAD, D), jnp.float32),
    )(s2, cnt, h, w2r_t, b2)


def kernel(x, edge_index, W1_l, b1_l, W1_r, W2_l, b2_l, W2_r):
    src = edge_index[0]
    dst = edge_index[1]
    pad = EPAD - E
    src_p = jnp.concatenate([src, jnp.zeros((pad,), jnp.int32)])
    dst_p = jnp.concatenate(
        [dst, N + (jnp.arange(pad, dtype=jnp.int32) % (NPAD - N))])
    x_pad = jnp.pad(x, ((0, NPAD - N), (0, 0)))
    b1r = b1_l.reshape(1, D)
    b2r = b2_l.reshape(1, D)

    y1 = _tc_matmul(x_pad, W1_l.T)
    s1, cnt = _sc_segment_sum(y1, src_p, dst_p, True)
    cnt = cnt.reshape(NC, NPAD, 1)
    h, y2 = _tc_mid(s1, cnt, x_pad, W1_r.T, b1r, W2_l.T)
    s2 = _sc_segment_sum(y2, src_p, dst_p, False)
    out = _tc_fin(s2, cnt, h, W2_r.T, b2r)
    return out[:N]


# speedup vs baseline: 1.3740x; 1.3740x over previous
import functools

import jax
import jax.numpy as jnp
from jax import lax
from jax.experimental import pallas as pl
from jax.experimental.pallas import tpu as pltpu
from jax.experimental.pallas import tpu_sc as plsc

N = 10000
E = 320000
D = 128

NC = 2
NS = 16
NW = NC * NS
NPAD = 10240
K = 128
CH_PAIR = 160
EPAD = NS * CH_PAIR * K
N0 = 113
N1 = CH_PAIR - N0
PB = 2
GLA = 1
IDXB = 4
ILA = 3
RPT = NPAD // NS


def _sc_segment_body(with_cnt, y_hbm, src_hbm, dst_hbm, *refs):
    if with_cnt:
        (s_hbm, cnt_hbm,
         acc, cntacc, sidx, didx, rows, ones, zcnt, gsem, ssem, isem) = refs
    else:
        (s_hbm,
         acc, sidx, didx, rows, gsem, ssem, isem) = refs
    cid = lax.axis_index("c")
    sid = lax.axis_index("s")
    myrows = pl.ds(sid * RPT, RPT)

    z16 = jnp.zeros((16,), jnp.float32)

    def _zrow(i, _):
        rows[0, i // (D // 16), pl.ds(lax.rem(i, D // 16) * 16, 16)] = z16
        return 0

    lax.fori_loop(0, K * D // 16, _zrow, 0)
    for j in range(RPT // K):
        pltpu.sync_copy(rows.at[0], acc.at[pl.ds(sid * RPT + j * K, K)])
    rem_rows = RPT - (RPT // K) * K
    if rem_rows:
        pltpu.sync_copy(rows.at[0, pl.ds(0, rem_rows)],
                        acc.at[pl.ds(sid * RPT + (RPT // K) * K, rem_rows)])
    if with_cnt:
        def _zone(i, _):
            ones[pl.ds(i * 16, 16)] = jnp.ones((16,), jnp.float32)
            return 0

        lax.fori_loop(0, K // 16, _zone, 0)

        def _zc(i, _):
            zcnt[pl.ds(i * 16, 16)] = z16
            return 0

        lax.fori_loop(0, RPT // 16, _zc, 0)
        pltpu.sync_copy(zcnt, cntacc.at[myrows])
    plsc.subcore_barrier()

    nch = jnp.where(cid == 0, N0, N1)
    ebase = (sid * CH_PAIR + jnp.where(cid == 0, 0, N0)) * K

    def _idx(c):
        ib = lax.rem(c, IDXB)
        off = ebase + c * K
        pltpu.async_copy(src_hbm.at[pl.ds(off, K)], sidx.at[ib],
                         isem.at[ib])
        pltpu.async_copy(dst_hbm.at[pl.ds(off, K)], didx.at[ib],
                         isem.at[ib])

    def _wait_idx(c):
        ib = lax.rem(c, IDXB)
        off = ebase + c * K
        pltpu.make_async_copy(src_hbm.at[pl.ds(off, K)], sidx.at[ib],
                              isem.at[ib]).wait()
        pltpu.make_async_copy(dst_hbm.at[pl.ds(off, K)], didx.at[ib],
                              isem.at[ib]).wait()

    def _gather(c):
        pltpu.async_copy(y_hbm.at[sidx.at[lax.rem(c, IDXB)]],
                         rows.at[lax.rem(c, PB)], gsem.at[lax.rem(c, PB)])

    def _wait_gather(c):
        pltpu.make_async_copy(y_hbm.at[sidx.at[lax.rem(c, IDXB)]],
                              rows.at[lax.rem(c, PB)],
                              gsem.at[lax.rem(c, PB)]).wait()

    def _scatter(c):
        b = lax.rem(c, PB)
        ib = lax.rem(c, IDXB)
        pltpu.async_copy(rows.at[b], acc.at[didx.at[ib]], ssem.at[b],
                         add=True)
        if with_cnt:
            pltpu.async_copy(ones, cntacc.at[didx.at[ib]], ssem.at[b],
                             add=True)

    def _wait_scatter(c):
        b = lax.rem(c, PB)
        ib = lax.rem(c, IDXB)
        pltpu.make_async_copy(rows.at[b], acc.at[didx.at[ib]],
                              ssem.at[b]).wait()
        if with_cnt:
            pltpu.make_async_copy(ones, cntacc.at[didx.at[ib]],
                                  ssem.at[b]).wait()

    for i in range(ILA):
        @pl.when(i < nch)
        def _():
            _idx(i)
    for i in range(GLA):
        @pl.when(i < nch)
        def _():
            _wait_idx(i)
            _gather(i)

    def _chunk(c, _):
        @pl.when(c >= 1)
        def _():
            _wait_scatter(c - 1)

        @pl.when(c + ILA < nch)
        def _():
            _idx(c + ILA)

        @pl.when(c + GLA < nch)
        def _():
            _wait_idx(c + GLA)
            _gather(c + GLA)

        _wait_gather(c)
        _scatter(c)
        return 0

    lax.fori_loop(0, nch, _chunk, 0)

    @pl.when(nch >= 1)
    def _():
        _wait_scatter(nch - 1)
    plsc.subcore_barrier()

    pltpu.sync_copy(acc.at[myrows], s_hbm.at[pl.ds(cid * NPAD + sid * RPT,
                                                   RPT)])
    if with_cnt:
        pltpu.sync_copy(cntacc.at[myrows],
                        cnt_hbm.at[pl.ds(cid * NPAD + sid * RPT, RPT)])


def _sc_segment_sum(y, src, dst, with_cnt):
    mesh = plsc.VectorSubcoreMesh(core_axis_name="c", subcore_axis_name="s")
    if with_cnt:
        out_type = (
            pltpu.HBM((NC * NPAD, D), jnp.float32),
            pltpu.HBM((NC * NPAD,), jnp.float32),
        )
        scratch = [
            pltpu.VMEM_SHARED((NPAD, D), jnp.float32),
            pltpu.VMEM_SHARED((NPAD,), jnp.float32),
            pltpu.VMEM((IDXB, K), jnp.int32),
            pltpu.VMEM((IDXB, K), jnp.int32),
            pltpu.VMEM((PB, K, D), jnp.float32),
            pltpu.VMEM((K,), jnp.float32),
            pltpu.VMEM((RPT,), jnp.float32),
            pltpu.SemaphoreType.DMA((PB,)),
            pltpu.SemaphoreType.DMA((PB,)),
            pltpu.SemaphoreType.DMA((IDXB,)),
        ]
        args = (y, src, dst)
    else:
        out_type = pltpu.HBM((NC * NPAD, D), jnp.float32)
        scratch = [
            pltpu.VMEM_SHARED((NPAD, D), jnp.float32),
            pltpu.VMEM((IDXB, K), jnp.int32),
            pltpu.VMEM((IDXB, K), jnp.int32),
            pltpu.VMEM((PB, K, D), jnp.float32),
            pltpu.SemaphoreType.DMA((PB,)),
            pltpu.SemaphoreType.DMA((PB,)),
            pltpu.SemaphoreType.DMA((IDXB,)),
        ]
        args = (y, src, dst)
    return pl.kernel(
        functools.partial(_sc_segment_body, with_cnt),
        out_type=out_type,
        mesh=mesh,
        scratch_types=scratch,
    )(*args)


def _mm_body(x_ref, w_ref, o_ref):
    o_ref[...] = jnp.dot(x_ref[...], w_ref[...],
                         preferred_element_type=jnp.float32)


def _tc_matmul(x, wt):
    return pl.pallas_call(
        _mm_body,
        out_shape=jax.ShapeDtypeStruct((NPAD, D), jnp.float32),
    )(x, wt)


def _mid_body(s_ref, c_ref, x_ref, wr_ref, b_ref, wl2_ref, h_ref, y2_ref):
    s = s_ref[0] + s_ref[1]
    c = c_ref[0] + c_ref[1]
    agg = s / jnp.maximum(c, 1.0)
    h = jnp.maximum(
        agg + b_ref[...]
        + jnp.dot(x_ref[...], wr_ref[...], preferred_element_type=jnp.float32),
        0.0)
    h_ref[...] = h
    y2_ref[...] = jnp.dot(h, wl2_ref[...], preferred_element_type=jnp.float32)


def _tc_mid(s1, cnt, x, w1r_t, b1, w2l_t):
    return pl.pallas_call(
        _mid_body,
        out_shape=(
            jax.ShapeDtypeStruct((NPAD, D), jnp.float32),
            jax.ShapeDtypeStruct((NPAD, D), jnp.float32),
        ),
    )(s1, cnt, x, w1r_t, b1, w2l_t)


def _fin_body(s_ref, c_ref, h_ref, wr_ref, b_ref, o_ref):
    s = s_ref[0] + s_ref[1]
    c = c_ref[0] + c_ref[1]
    agg = s / jnp.maximum(c, 1.0)
    o_ref[...] = jnp.maximum(
        agg + b_ref[...]
        + jnp.dot(h_ref[...], wr_ref[...], preferred_element_type=jnp.float32),
        0.0)


def _tc_fin(s2, cnt, h, w2r_t, b2):
    return pl.pallas_call(
        _fin_body,
        out_shape=jax.ShapeDtypeStruct((NPAD, D), jnp.float32),
    )(s2, cnt, h, w2r_t, b2)


def kernel(x, edge_index, W1_l, b1_l, W1_r, W2_l, b2_l, W2_r):
    src = edge_index[0]
    dst = edge_index[1]
    pad = EPAD - E
    src_p = jnp.concatenate([src, jnp.zeros((pad,), jnp.int32)])
    dst_p = jnp.concatenate(
        [dst, N + (jnp.arange(pad, dtype=jnp.int32) % (NPAD - N))])
    x_pad = jnp.pad(x, ((0, NPAD - N), (0, 0)))
    b1r = b1_l.reshape(1, D)
    b2r = b2_l.reshape(1, D)

    y1 = _tc_matmul(x_pad, W1_l.T)
    s1, cnt = _sc_segment_sum(y1, src_p, dst_p, True)
    s1 = s1.reshape(NC, NPAD, D)
    cnt = cnt.reshape(NC, NPAD, 1)
    h, y2 = _tc_mid(s1, cnt, x_pad, W1_r.T, b1r, W2_l.T)
    s2 = _sc_segment_sum(y2, src_p, dst_p, False).reshape(NC, NPAD, D)
    out = _tc_fin(s2, cnt, h, W2_r.T, b2r)
    return out[:N]
